# Initial kernel scaffold; baseline (speedup 1.0000x reference)
import jax
import jax.numpy as jnp
from jax.experimental import pallas as pl


def kernel(query_xyz, support_xyz, query_mask, support_mask, support_features, W_conv, b_conv, bn_gamma, bn_beta):
    raise NotImplementedError("write your pallas kernel here")


# speedup vs baseline: 8.6753x; 8.6753x over previous
import functools
import math

import jax
import jax.numpy as jnp
from jax import lax
from jax.experimental import pallas as pl
from jax.experimental.pallas import tpu as pltpu
from jax.experimental.pallas import tpu_sc as plsc

RADIUS = 0.1
NSAMPLE = 32
LANES = 16
NB = 512


def _prep_body(feats_ref, sxyz_ref, qxyz_ref, p_ref, t_ref, d_ref):
    s = sxyz_ref[...]
    q = qxyz_ref[...]
    p = p_ref[...]
    a = (s[:, 0:1] * p[0:1, :] + s[:, 1:2] * p[1:2, :]
         + s[:, 2:3] * p[2:3, :] + p[3:4, :])
    d = (q[:, 0:1] * p[0:1, :] + q[:, 1:2] * p[1:2, :]
         + q[:, 2:3] * p[2:3, :])
    ft = jnp.transpose(feats_ref[...], (1, 0))
    t_ref[...] = jnp.concatenate([ft * a, ft], axis=1)
    d_ref[...] = jnp.concatenate(
        [d, q, jnp.zeros((q.shape[0], 61), jnp.float32)], axis=1)


def _prep_call(feats_flat, sxyz_flat, qxyz_flat, params, B, N1, N2, C):
    nblk = N2 // NB
    return pl.pallas_call(
        _prep_body,
        grid=(B, nblk),
        in_specs=[
            pl.BlockSpec((C, NB), lambda b, n: (b, n)),
            pl.BlockSpec((NB, 3), lambda b, n: (b * nblk + n, 0)),
            pl.BlockSpec((NB, 3), lambda b, n: (b * nblk + n, 0)),
            pl.BlockSpec((8, 64), lambda b, n: (0, 0)),
        ],
        out_specs=[
            pl.BlockSpec((NB, 128), lambda b, n: (b * nblk + n, 0)),
            pl.BlockSpec((NB, 128), lambda b, n: (b * nblk + n, 0)),
        ],
        out_shape=[
            jax.ShapeDtypeStruct((B * N2, 128), jnp.float32),
            jax.ShapeDtypeStruct((B * N1, 128), jnp.float32),
        ],
    )(feats_flat, sxyz_flat, qxyz_flat, params)


def _post_body(x_ref, p2_ref, o_ref):
    x = x_ref[...]
    y = jnp.maximum(x * p2_ref[0:1, :] + p2_ref[1:2, :], 0.0)
    o_ref[...] = jnp.transpose(y, (1, 0))


def _post_call(raw, params2, B, N1, C):
    nblk = N1 // NB
    return pl.pallas_call(
        _post_body,
        grid=(B, nblk),
        in_specs=[
            pl.BlockSpec((NB, C), lambda b, n: (b * nblk + n, 0)),
            pl.BlockSpec((8, 64), lambda b, n: (0, 0)),
        ],
        out_specs=pl.BlockSpec((C, NB), lambda b, n: (b, n)),
        out_shape=jax.ShapeDtypeStruct((B * C, N1), jnp.float32),
    )(raw, params2)


def _sc_call(sxyz_t, dq_flat, t_flat, B, N1, N2):
    info = plsc.get_sparse_core_info()
    nc, ns = info.num_cores, info.num_subcores
    nw = nc * ns
    qpw = (B * N1) // nw
    wpb = nw // B
    nchunk = N2 // LANES
    r2 = RADIUS * RADIUS
    mesh = plsc.VectorSubcoreMesh(core_axis_name="c", subcore_axis_name="s")

    @functools.partial(
        pl.kernel,
        mesh=mesh,
        out_type=jax.ShapeDtypeStruct((B * N1 // 2, 128), jnp.float32),
        compiler_params=pltpu.CompilerParams(needs_layout_passes=False),
        scratch_types=[
            pltpu.VMEM((8, N2), jnp.float32),
            pltpu.VMEM((qpw, 128), jnp.float32),
            pltpu.VMEM((64,), jnp.int32),
            pltpu.VMEM((NSAMPLE,), jnp.int32),
            pltpu.VMEM((NSAMPLE, 128), jnp.float32),
            pltpu.VMEM((qpw // 2, 128), jnp.float32),
            pltpu.SemaphoreType.DMA,
        ],
    )
    def kern(sxyz_hbm, dq_hbm, t_hbm, out_hbm,
             sxyz_v, dq_v, list_v, idx_v, rows_v, out_v, sem):
        wid = lax.axis_index("s") * nc + lax.axis_index("c")
        base = wid * qpw
        batch = wid // wpb
        pltpu.sync_copy(sxyz_hbm.at[pl.ds(batch * 8, 8)], sxyz_v)
        pltpu.sync_copy(dq_hbm.at[pl.ds(base, qpw)], dq_v)

        lane_iota = lax.iota(jnp.int32, LANES)
        zero16 = jnp.zeros((LANES,), jnp.int32)
        tbase = batch * N2

        def per_query(i, carry):
            qv = dq_v[i, pl.ds(64, LANES)]
            qx = qv[0]
            qy = qv[1]
            qz = qv[2]
            list_v[pl.ds(0, LANES)] = zero16
            list_v[pl.ds(LANES, LANES)] = zero16

            def chunk(t, cnt):
                off = t * LANES
                dx = sxyz_v[0, pl.ds(off, LANES)] - qx
                dy = sxyz_v[1, pl.ds(off, LANES)] - qy
                dz = sxyz_v[2, pl.ds(off, LANES)] - qz
                d2 = (dx * dx + dy * dy) + dz * dz
                m = d2 < r2

                @pl.when(cnt < NSAMPLE)
                def _():
                    plsc.store_compressed(
                        list_v.at[pl.ds(cnt, LANES)], lane_iota + off, mask=m)

                return cnt + jnp.sum(m.astype(jnp.int32))

            cnt = lax.fori_loop(0, nchunk, chunk, 0)

            first = list_v[pl.ds(0, LANES)][0]
            for h in range(2):
                cur = list_v[pl.ds(h * LANES, LANES)]
                lane = lane_iota + h * LANES
                idx_v[pl.ds(h * LANES, LANES)] = (
                    jnp.where(lane < cnt, cur, first) + tbase)

            pltpu.async_copy(t_hbm.at[idx_v], rows_v, sem).wait()

            dvec = [dq_v[i, pl.ds(h * LANES, LANES)] for h in range(4)]
            acc = [rows_v[0, pl.ds(h * LANES, LANES)]
                   - rows_v[0, pl.ds(64 + h * LANES, LANES)] * dvec[h]
                   for h in range(4)]
            for j in range(1, NSAMPLE):
                for h in range(4):
                    pj = rows_v[j, pl.ds(h * LANES, LANES)]
                    fj = rows_v[j, pl.ds(64 + h * LANES, LANES)]
                    acc[h] = jnp.maximum(acc[h], pj - fj * dvec[h])
            half = (i & 1) * 64
            for h in range(4):
                out_v[i // 2, pl.ds(half + h * LANES, LANES)] = acc[h]
            return carry

        lax.fori_loop(0, qpw, per_query, 0)
        pltpu.sync_copy(out_v, out_hbm.at[pl.ds(wid * (qpw // 2), qpw // 2)])

    return kern(sxyz_t, dq_flat, t_flat)


def kernel(query_xyz, support_xyz, query_mask, support_mask, support_features,
           W_conv, b_conv, bn_gamma, bn_beta):
    B, C, N2 = support_features.shape
    N1 = query_xyz.shape[1]

    params = jnp.zeros((8, 64), jnp.float32)
    params = params.at[0:3].set(W_conv.T / RADIUS).at[3].set(b_conv)
    params2 = jnp.zeros((8, 64), jnp.float32)
    params2 = params2.at[0].set(bn_gamma * (1.0 / math.sqrt(1.0 + 1e-5)))
    params2 = params2.at[1].set(bn_beta)

    feats_flat = support_features.reshape(B * C, N2)
    sxyz_flat = support_xyz.reshape(B * N2, 3)
    qxyz_flat = query_xyz.reshape(B * N1, 3)

    t_flat, dq_flat = _prep_call(feats_flat, sxyz_flat, qxyz_flat, params,
                                 B, N1, N2, C)

    sxyz_t = jnp.pad(jnp.swapaxes(support_xyz, 1, 2),
                     ((0, 0), (0, 5), (0, 0))).reshape(B * 8, N2)

    raw = _sc_call(sxyz_t, dq_flat, t_flat, B, N1, N2)
    raw = raw.reshape(B * N1, 64)
    out = _post_call(raw, params2, B, N1, C)
    return out.reshape(B, C, N1)


# speedup vs baseline: 16.6673x; 1.9212x over previous
import functools
import math

import jax
import jax.numpy as jnp
from jax import lax
from jax.experimental import pallas as pl
from jax.experimental.pallas import tpu as pltpu
from jax.experimental.pallas import tpu_sc as plsc

RADIUS = 0.1
NSAMPLE = 32
LANES = 16
NB = 512


def _prep_body(feats_ref, sxyz_ref, qxyz_ref, p_ref, t_ref, d_ref):
    s = sxyz_ref[...]
    q = qxyz_ref[...]
    p = p_ref[...]
    a = (s[:, 0:1] * p[0:1, :] + s[:, 1:2] * p[1:2, :]
         + s[:, 2:3] * p[2:3, :] + p[3:4, :])
    d = (q[:, 0:1] * p[0:1, :] + q[:, 1:2] * p[1:2, :]
         + q[:, 2:3] * p[2:3, :])
    ft = jnp.transpose(feats_ref[...], (1, 0))
    t_ref[...] = jnp.concatenate([ft * a, ft], axis=1)
    d_ref[...] = jnp.concatenate(
        [d, q, jnp.zeros((q.shape[0], 61), jnp.float32)], axis=1)


def _prep_call(feats_flat, sxyz_flat, qxyz_flat, params, B, N1, N2, C):
    nblk = N2 // NB
    return pl.pallas_call(
        _prep_body,
        grid=(B, nblk),
        in_specs=[
            pl.BlockSpec((C, NB), lambda b, n: (b, n)),
            pl.BlockSpec((NB, 3), lambda b, n: (b * nblk + n, 0)),
            pl.BlockSpec((NB, 3), lambda b, n: (b * nblk + n, 0)),
            pl.BlockSpec((8, 64), lambda b, n: (0, 0)),
        ],
        out_specs=[
            pl.BlockSpec((NB, 128), lambda b, n: (b * nblk + n, 0)),
            pl.BlockSpec((NB, 128), lambda b, n: (b * nblk + n, 0)),
        ],
        out_shape=[
            jax.ShapeDtypeStruct((B * N2, 128), jnp.float32),
            jax.ShapeDtypeStruct((B * N1, 128), jnp.float32),
        ],
    )(feats_flat, sxyz_flat, qxyz_flat, params)


MB = 256


def _mask_body(qxyz_ref, sxyzt_ref, g_ref, w_ref):
    r2 = RADIUS * RADIUS
    q = qxyz_ref[...]
    dx = q[:, 0:1] - sxyzt_ref[0:1, :]
    dy = q[:, 1:2] - sxyzt_ref[1:2, :]
    dz = q[:, 2:3] - sxyzt_ref[2:3, :]
    d2 = (dx * dx + dy * dy) + dz * dz
    mb = jnp.where(d2 < r2, 1.0, 0.0).astype(jnp.bfloat16)
    words = jax.lax.dot_general(mb, g_ref[...], (((1,), (0,)), ((), ())),
                                preferred_element_type=jnp.float32)
    w_ref[...] = words.astype(jnp.int32)


def _mask_call(qxyz_flat, sxyz_t, gmat, B, N1, N2):
    nblk = N1 // MB
    nh = N2 // LANES
    return pl.pallas_call(
        _mask_body,
        grid=(B, nblk),
        in_specs=[
            pl.BlockSpec((MB, 3), lambda b, n: (b * nblk + n, 0)),
            pl.BlockSpec((8, N2), lambda b, n: (b, 0)),
            pl.BlockSpec((N2, nh), lambda b, n: (0, 0)),
        ],
        out_specs=pl.BlockSpec((MB, nh), lambda b, n: (b * nblk + n, 0)),
        out_shape=jax.ShapeDtypeStruct((B * N1, nh), jnp.int32),
    )(qxyz_flat, sxyz_t, gmat)


def _post_body(x_ref, p2_ref, o_ref):
    x = x_ref[...]
    y = jnp.maximum(x * p2_ref[0:1, :] + p2_ref[1:2, :], 0.0)
    o_ref[...] = jnp.transpose(y, (1, 0))


def _post_call(raw, params2, B, N1, C):
    nblk = N1 // NB
    return pl.pallas_call(
        _post_body,
        grid=(B, nblk),
        in_specs=[
            pl.BlockSpec((NB, C), lambda b, n: (b * nblk + n, 0)),
            pl.BlockSpec((8, 64), lambda b, n: (0, 0)),
        ],
        out_specs=pl.BlockSpec((C, NB), lambda b, n: (b, n)),
        out_shape=jax.ShapeDtypeStruct((B * C, N1), jnp.float32),
    )(raw, params2)


def _sc_call(words, dq_flat, t_flat, B, N1, N2):
    info = plsc.get_sparse_core_info()
    nc, ns = info.num_cores, info.num_subcores
    nw = nc * ns
    qpw = (B * N1) // nw
    wpb = nw // B
    nh = N2 // LANES
    mesh = plsc.VectorSubcoreMesh(core_axis_name="c", subcore_axis_name="s")

    @functools.partial(
        pl.kernel,
        mesh=mesh,
        out_type=jax.ShapeDtypeStruct((B * N1 // 2, 128), jnp.float32),
        compiler_params=pltpu.CompilerParams(needs_layout_passes=False),
        scratch_types=[
            pltpu.VMEM((qpw, nh), jnp.int32),
            pltpu.VMEM((qpw, 128), jnp.float32),
            pltpu.VMEM((nh + LANES,), jnp.int32),
            pltpu.VMEM((64,), jnp.int32),
            pltpu.VMEM((NSAMPLE,), jnp.int32),
            pltpu.VMEM((NSAMPLE, 128), jnp.float32),
            pltpu.VMEM((qpw // 2, 128), jnp.float32),
            pltpu.SemaphoreType.DMA,
        ],
    )
    def kern(words_hbm, dq_hbm, t_hbm, out_hbm,
             words_v, dq_v, pos_v, list_v, idx_v, rows_v, out_v, sem):
        wid = lax.axis_index("s") * nc + lax.axis_index("c")
        base = wid * qpw
        batch = wid // wpb
        pltpu.sync_copy(words_hbm.at[pl.ds(base, qpw)], words_v)
        pltpu.sync_copy(dq_hbm.at[pl.ds(base, qpw)], dq_v)

        lane_iota = lax.iota(jnp.int32, LANES)
        zero16 = jnp.zeros((LANES,), jnp.int32)
        tbase = batch * N2

        def per_query(i, carry):
            list_v[pl.ds(0, LANES)] = zero16
            list_v[pl.ds(LANES, LANES)] = zero16

            def wgroup(g, wcnt):
                w = words_v[i, pl.ds(g * LANES, LANES)]
                m = w != 0
                plsc.store_compressed(
                    pos_v.at[pl.ds(wcnt, LANES)], lane_iota + g * LANES, mask=m)
                return wcnt + jnp.sum(m.astype(jnp.int32))

            wcnt = lax.fori_loop(0, nh // LANES, wgroup, 0)

            ivec = jnp.full((LANES,), i, jnp.int32)

            def bitword(state):
                j, cnt = state
                posvec = plsc.load_gather(pos_v, [jnp.full((LANES,), j, jnp.int32)])
                wvec = plsc.load_gather(words_v, [ivec, posvec])
                m = ((wvec >> lane_iota) & 1) != 0
                plsc.store_compressed(
                    list_v.at[pl.ds(cnt, LANES)],
                    posvec * LANES + lane_iota, mask=m)
                return j + 1, cnt + jnp.sum(m.astype(jnp.int32))

            _, cnt = lax.while_loop(
                lambda s: (s[0] < wcnt) & (s[1] < NSAMPLE), bitword, (0, 0))

            first = list_v[pl.ds(0, LANES)][0]
            for h in range(2):
                cur = list_v[pl.ds(h * LANES, LANES)]
                lane = lane_iota + h * LANES
                idx_v[pl.ds(h * LANES, LANES)] = (
                    jnp.where(lane < cnt, cur, first) + tbase)

            pltpu.async_copy(t_hbm.at[idx_v], rows_v, sem).wait()

            dvec = [dq_v[i, pl.ds(h * LANES, LANES)] for h in range(4)]
            acc = [rows_v[0, pl.ds(h * LANES, LANES)]
                   - rows_v[0, pl.ds(64 + h * LANES, LANES)] * dvec[h]
                   for h in range(4)]
            for j in range(1, NSAMPLE):
                for h in range(4):
                    pj = rows_v[j, pl.ds(h * LANES, LANES)]
                    fj = rows_v[j, pl.ds(64 + h * LANES, LANES)]
                    acc[h] = jnp.maximum(acc[h], pj - fj * dvec[h])
            half = (i & 1) * 64
            for h in range(4):
                out_v[i // 2, pl.ds(half + h * LANES, LANES)] = acc[h]
            return carry

        lax.fori_loop(0, qpw, per_query, 0)
        pltpu.sync_copy(out_v, out_hbm.at[pl.ds(wid * (qpw // 2), qpw // 2)])

    return kern(words, dq_flat, t_flat)


def kernel(query_xyz, support_xyz, query_mask, support_mask, support_features,
           W_conv, b_conv, bn_gamma, bn_beta):
    B, C, N2 = support_features.shape
    N1 = query_xyz.shape[1]

    params = jnp.zeros((8, 64), jnp.float32)
    params = params.at[0:3].set(W_conv.T / RADIUS).at[3].set(b_conv)
    params2 = jnp.zeros((8, 64), jnp.float32)
    params2 = params2.at[0].set(bn_gamma * (1.0 / math.sqrt(1.0 + 1e-5)))
    params2 = params2.at[1].set(bn_beta)

    feats_flat = support_features.reshape(B * C, N2)
    sxyz_flat = support_xyz.reshape(B * N2, 3)
    qxyz_flat = query_xyz.reshape(B * N1, 3)

    t_flat, dq_flat = _prep_call(feats_flat, sxyz_flat, qxyz_flat, params,
                                 B, N1, N2, C)

    sxyz_t = jnp.pad(jnp.swapaxes(support_xyz, 1, 2),
                     ((0, 0), (0, 5), (0, 0))).reshape(B * 8, N2)
    s_ar = jnp.arange(N2)
    gmat = jnp.where(
        (s_ar[:, None] // LANES) == jnp.arange(N2 // LANES)[None, :],
        (2.0 ** (s_ar % LANES))[:, None], 0.0).astype(jnp.bfloat16)

    words = _mask_call(qxyz_flat, sxyz_t, gmat, B, N1, N2)
    raw = _sc_call(words, dq_flat, t_flat, B, N1, N2)
    raw = raw.reshape(B * N1, 64)
    out = _post_call(raw, params2, B, N1, C)
    return out.reshape(B, C, N1)


# speedup vs baseline: 24.2246x; 1.4534x over previous
import functools
import math

import jax
import jax.numpy as jnp
from jax import lax
from jax.experimental import pallas as pl
from jax.experimental.pallas import tpu as pltpu
from jax.experimental.pallas import tpu_sc as plsc

RADIUS = 0.1
NSAMPLE = 32
LANES = 16
NB = 512


def _prep_body(feats_ref, sxyz_ref, qxyz_ref, p_ref, t_ref, d_ref):
    s = sxyz_ref[...]
    q = qxyz_ref[...]
    p = p_ref[...]
    a = (s[:, 0:1] * p[0:1, :] + s[:, 1:2] * p[1:2, :]
         + s[:, 2:3] * p[2:3, :] + p[3:4, :])
    d = (q[:, 0:1] * p[0:1, :] + q[:, 1:2] * p[1:2, :]
         + q[:, 2:3] * p[2:3, :])
    ft = jnp.transpose(feats_ref[...], (1, 0))
    t_ref[...] = jnp.concatenate([ft * a, ft], axis=1)
    d_ref[...] = jnp.concatenate(
        [d, q, jnp.zeros((q.shape[0], 61), jnp.float32)], axis=1)


def _prep_call(feats_flat, sxyz_flat, qxyz_flat, params, B, N1, N2, C):
    nblk = N2 // NB
    return pl.pallas_call(
        _prep_body,
        grid=(B, nblk),
        in_specs=[
            pl.BlockSpec((C, NB), lambda b, n: (b, n)),
            pl.BlockSpec((NB, 3), lambda b, n: (b * nblk + n, 0)),
            pl.BlockSpec((NB, 3), lambda b, n: (b * nblk + n, 0)),
            pl.BlockSpec((8, 64), lambda b, n: (0, 0)),
        ],
        out_specs=[
            pl.BlockSpec((NB, 128), lambda b, n: (b * nblk + n, 0)),
            pl.BlockSpec((NB, 128), lambda b, n: (b * nblk + n, 0)),
        ],
        out_shape=[
            jax.ShapeDtypeStruct((B * N2, 128), jnp.float32),
            jax.ShapeDtypeStruct((B * N1, 128), jnp.float32),
        ],
    )(feats_flat, sxyz_flat, qxyz_flat, params)


MB = 256


def _mask_body(qxyz_ref, sxyzt_ref, g_ref, w_ref):
    r2 = RADIUS * RADIUS
    q = qxyz_ref[...]
    dx = q[:, 0:1] - sxyzt_ref[0:1, :]
    dy = q[:, 1:2] - sxyzt_ref[1:2, :]
    dz = q[:, 2:3] - sxyzt_ref[2:3, :]
    d2 = (dx * dx + dy * dy) + dz * dz
    mb = jnp.where(d2 < r2, 1.0, 0.0).astype(jnp.bfloat16)
    words = jax.lax.dot_general(mb, g_ref[...], (((1,), (0,)), ((), ())),
                                preferred_element_type=jnp.float32)
    w_ref[...] = words.astype(jnp.int32)


def _mask_call(qxyz_flat, sxyz_t, gmat, B, N1, N2):
    nblk = N1 // MB
    nh = N2 // LANES
    return pl.pallas_call(
        _mask_body,
        grid=(B, nblk),
        in_specs=[
            pl.BlockSpec((MB, 3), lambda b, n: (b * nblk + n, 0)),
            pl.BlockSpec((8, N2), lambda b, n: (b, 0)),
            pl.BlockSpec((N2, nh), lambda b, n: (0, 0)),
        ],
        out_specs=pl.BlockSpec((MB, nh), lambda b, n: (b * nblk + n, 0)),
        out_shape=jax.ShapeDtypeStruct((B * N1, nh), jnp.int32),
    )(qxyz_flat, sxyz_t, gmat)


def _post_body(x_ref, p2_ref, o_ref):
    x = x_ref[...]
    y = jnp.maximum(x * p2_ref[0:1, :] + p2_ref[1:2, :], 0.0)
    o_ref[...] = jnp.transpose(y, (1, 0))


def _post_call(raw, params2, B, N1, C):
    nblk = N1 // NB
    return pl.pallas_call(
        _post_body,
        grid=(B, nblk),
        in_specs=[
            pl.BlockSpec((NB, C), lambda b, n: (b * nblk + n, 0)),
            pl.BlockSpec((8, 64), lambda b, n: (0, 0)),
        ],
        out_specs=pl.BlockSpec((C, NB), lambda b, n: (b, n)),
        out_shape=jax.ShapeDtypeStruct((B * C, N1), jnp.float32),
    )(raw, params2)


def _sc_call(words, dq_flat, t_flat, B, N1, N2):
    info = plsc.get_sparse_core_info()
    nc, ns = info.num_cores, info.num_subcores
    nw = nc * ns
    qpw = (B * N1) // nw
    wpb = nw // B
    nh = N2 // LANES
    mesh = plsc.VectorSubcoreMesh(core_axis_name="c", subcore_axis_name="s")

    @functools.partial(
        pl.kernel,
        mesh=mesh,
        out_type=jax.ShapeDtypeStruct((B * N1 // 2, 128), jnp.float32),
        compiler_params=pltpu.CompilerParams(needs_layout_passes=False),
        scratch_types=[
            pltpu.VMEM((qpw, nh), jnp.int32),
            pltpu.VMEM((qpw, 128), jnp.float32),
            pltpu.VMEM((nh + LANES,), jnp.int32),
            pltpu.VMEM((64,), jnp.int32),
            pltpu.VMEM((NSAMPLE,), jnp.int32),
            pltpu.VMEM((NSAMPLE,), jnp.int32),
            pltpu.VMEM((NSAMPLE, 128), jnp.float32),
            pltpu.VMEM((NSAMPLE, 128), jnp.float32),
            pltpu.VMEM((qpw // 2, 128), jnp.float32),
            pltpu.SemaphoreType.DMA,
            pltpu.SemaphoreType.DMA,
        ],
    )
    def kern(words_hbm, dq_hbm, t_hbm, out_hbm,
             words_v, dq_v, pos_v, list_v, idx0_v, idx1_v, rows0_v, rows1_v,
             out_v, sem0, sem1):
        wid = lax.axis_index("s") * nc + lax.axis_index("c")
        base = wid * qpw
        batch = wid // wpb
        pltpu.sync_copy(words_hbm.at[pl.ds(base, qpw)], words_v)
        pltpu.sync_copy(dq_hbm.at[pl.ds(base, qpw)], dq_v)

        lane_iota = lax.iota(jnp.int32, LANES)
        zero16 = jnp.zeros((LANES,), jnp.int32)
        tbase = batch * N2

        def popcnt(m):
            return plsc.all_reduce_population_count(m)[0]

        def scan_pad(i, idx_ref):
            list_v[pl.ds(0, LANES)] = zero16
            list_v[pl.ds(LANES, LANES)] = zero16

            def wgroup(g, wcnt):
                w = words_v[i, pl.ds(g * LANES, LANES)]
                m = w != 0
                plsc.store_compressed(
                    pos_v.at[pl.ds(wcnt, LANES)], lane_iota + g * LANES, mask=m)
                return wcnt + popcnt(m)

            wcnt = lax.fori_loop(0, nh // LANES, wgroup, 0, unroll=4)

            ivec = jnp.full((LANES,), i, jnp.int32)

            def bitword(state):
                j, cnt = state
                posvec = plsc.load_gather(
                    pos_v, [jnp.full((LANES,), j, jnp.int32)])
                wvec = plsc.load_gather(words_v, [ivec, posvec])
                m = ((wvec >> lane_iota) & 1) != 0
                plsc.store_compressed(
                    list_v.at[pl.ds(cnt, LANES)],
                    posvec * LANES + lane_iota, mask=m)
                return j + 1, cnt + popcnt(m)

            _, cnt = lax.while_loop(
                lambda s: (s[0] < wcnt) & (s[1] < NSAMPLE), bitword, (0, 0))

            first = list_v[pl.ds(0, LANES)][0]
            for h in range(2):
                cur = list_v[pl.ds(h * LANES, LANES)]
                lane = lane_iota + h * LANES
                idx_ref[pl.ds(h * LANES, LANES)] = (
                    jnp.where(lane < cnt, cur, first) + tbase)

        def compute(i, rows_v, half):
            dvec = [dq_v[i, pl.ds(h * LANES, LANES)] for h in range(4)]
            acc = [rows_v[0, pl.ds(h * LANES, LANES)]
                   - rows_v[0, pl.ds(64 + h * LANES, LANES)] * dvec[h]
                   for h in range(4)]
            for j in range(1, NSAMPLE):
                for h in range(4):
                    pj = rows_v[j, pl.ds(h * LANES, LANES)]
                    fj = rows_v[j, pl.ds(64 + h * LANES, LANES)]
                    acc[h] = jnp.maximum(acc[h], pj - fj * dvec[h])
            for h in range(4):
                out_v[i // 2, pl.ds(half + h * LANES, LANES)] = acc[h]

        def per_pair(k, carry):
            i0 = 2 * k
            i1 = 2 * k + 1
            scan_pad(i0, idx0_v)
            cp0 = pltpu.async_copy(t_hbm.at[idx0_v], rows0_v, sem0)
            scan_pad(i1, idx1_v)
            cp1 = pltpu.async_copy(t_hbm.at[idx1_v], rows1_v, sem1)
            cp0.wait()
            compute(i0, rows0_v, 0)
            cp1.wait()
            compute(i1, rows1_v, 64)
            return carry

        lax.fori_loop(0, qpw // 2, per_pair, 0)
        pltpu.sync_copy(out_v, out_hbm.at[pl.ds(wid * (qpw // 2), qpw // 2)])

    return kern(words, dq_flat, t_flat)


def kernel(query_xyz, support_xyz, query_mask, support_mask, support_features,
           W_conv, b_conv, bn_gamma, bn_beta):
    B, C, N2 = support_features.shape
    N1 = query_xyz.shape[1]

    params = jnp.zeros((8, 64), jnp.float32)
    params = params.at[0:3].set(W_conv.T / RADIUS).at[3].set(b_conv)
    params2 = jnp.zeros((8, 64), jnp.float32)
    params2 = params2.at[0].set(bn_gamma * (1.0 / math.sqrt(1.0 + 1e-5)))
    params2 = params2.at[1].set(bn_beta)

    feats_flat = support_features.reshape(B * C, N2)
    sxyz_flat = support_xyz.reshape(B * N2, 3)
    qxyz_flat = query_xyz.reshape(B * N1, 3)

    t_flat, dq_flat = _prep_call(feats_flat, sxyz_flat, qxyz_flat, params,
                                 B, N1, N2, C)

    sxyz_t = jnp.pad(jnp.swapaxes(support_xyz, 1, 2),
                     ((0, 0), (0, 5), (0, 0))).reshape(B * 8, N2)
    s_ar = jnp.arange(N2)
    gmat = jnp.where(
        (s_ar[:, None] // LANES) == jnp.arange(N2 // LANES)[None, :],
        (2.0 ** (s_ar % LANES))[:, None], 0.0).astype(jnp.bfloat16)

    words = _mask_call(qxyz_flat, sxyz_t, gmat, B, N1, N2)
    raw = _sc_call(words, dq_flat, t_flat, B, N1, N2)
    raw = raw.reshape(B * N1, 64)
    out = _post_call(raw, params2, B, N1, C)
    return out.reshape(B, C, N1)


# speedup vs baseline: 26.9812x; 1.1138x over previous
import functools
import math

import jax
import jax.numpy as jnp
from jax import lax
from jax.experimental import pallas as pl
from jax.experimental.pallas import tpu as pltpu
from jax.experimental.pallas import tpu_sc as plsc

RADIUS = 0.1
NSAMPLE = 32
LANES = 16
NB = 512


def _prep_body(feats_ref, sxyz_ref, qxyz_ref, p_ref, t_ref, d_ref):
    s = sxyz_ref[...]
    q = qxyz_ref[...]
    p = p_ref[...]
    a = (s[:, 0:1] * p[0:1, :] + s[:, 1:2] * p[1:2, :]
         + s[:, 2:3] * p[2:3, :] + p[3:4, :])
    d = (q[:, 0:1] * p[0:1, :] + q[:, 1:2] * p[1:2, :]
         + q[:, 2:3] * p[2:3, :])
    ft = jnp.transpose(feats_ref[...], (1, 0))
    t_ref[...] = jnp.concatenate([ft * a, ft], axis=1)
    d_ref[...] = jnp.concatenate(
        [d, q, jnp.zeros((q.shape[0], 61), jnp.float32)], axis=1)


def _prep_call(feats_flat, sxyz_flat, qxyz_flat, params, B, N1, N2, C):
    nblk = N2 // NB
    return pl.pallas_call(
        _prep_body,
        grid=(B, nblk),
        in_specs=[
            pl.BlockSpec((C, NB), lambda b, n: (b, n)),
            pl.BlockSpec((NB, 3), lambda b, n: (b * nblk + n, 0)),
            pl.BlockSpec((NB, 3), lambda b, n: (b * nblk + n, 0)),
            pl.BlockSpec((8, 64), lambda b, n: (0, 0)),
        ],
        out_specs=[
            pl.BlockSpec((NB, 128), lambda b, n: (b * nblk + n, 0)),
            pl.BlockSpec((NB, 128), lambda b, n: (b * nblk + n, 0)),
        ],
        out_shape=[
            jax.ShapeDtypeStruct((B * N2, 128), jnp.float32),
            jax.ShapeDtypeStruct((B * N1, 128), jnp.float32),
        ],
    )(feats_flat, sxyz_flat, qxyz_flat, params)


MB = 256


def _mask_body(qxyz_ref, sxyzt_ref, g_ref, gc_ref, w_ref):
    r2 = RADIUS * RADIUS
    q = qxyz_ref[...]
    dx = q[:, 0:1] - sxyzt_ref[0:1, :]
    dy = q[:, 1:2] - sxyzt_ref[1:2, :]
    dz = q[:, 2:3] - sxyzt_ref[2:3, :]
    d2 = (dx * dx + dy * dy) + dz * dz
    mb = jnp.where(d2 < r2, 1.0, 0.0).astype(jnp.bfloat16)
    dn = (((1,), (0,)), ((), ()))
    bits = jax.lax.dot_general(mb, g_ref[...], dn,
                               preferred_element_type=jnp.float32)
    cnts = jax.lax.dot_general(mb, gc_ref[...], dn,
                               preferred_element_type=jnp.float32)
    w_ref[...] = bits.astype(jnp.int32) + cnts.astype(jnp.int32) * 65536


def _mask_call(qxyz_flat, sxyz_t, gmat, gcmat, B, N1, N2):
    nblk = N1 // MB
    nh = N2 // LANES
    return pl.pallas_call(
        _mask_body,
        grid=(B, nblk),
        in_specs=[
            pl.BlockSpec((MB, 3), lambda b, n: (b * nblk + n, 0)),
            pl.BlockSpec((8, N2), lambda b, n: (b, 0)),
            pl.BlockSpec((N2, nh), lambda b, n: (0, 0)),
            pl.BlockSpec((N2, nh), lambda b, n: (0, 0)),
        ],
        out_specs=pl.BlockSpec((MB, nh), lambda b, n: (b * nblk + n, 0)),
        out_shape=jax.ShapeDtypeStruct((B * N1, nh), jnp.int32),
    )(qxyz_flat, sxyz_t, gmat, gcmat)


def _post_body(x_ref, p2_ref, o_ref):
    x = x_ref[...]
    y = jnp.maximum(x * p2_ref[0:1, :] + p2_ref[1:2, :], 0.0)
    o_ref[...] = jnp.transpose(y, (1, 0))


def _post_call(raw, params2, B, N1, C):
    nblk = N1 // NB
    return pl.pallas_call(
        _post_body,
        grid=(B, nblk),
        in_specs=[
            pl.BlockSpec((NB, C), lambda b, n: (b * nblk + n, 0)),
            pl.BlockSpec((8, 64), lambda b, n: (0, 0)),
        ],
        out_specs=pl.BlockSpec((C, NB), lambda b, n: (b, n)),
        out_shape=jax.ShapeDtypeStruct((B * C, N1), jnp.float32),
    )(raw, params2)


def _sc_call(words, dq_flat, t_flat, B, N1, N2):
    info = plsc.get_sparse_core_info()
    nc, ns = info.num_cores, info.num_subcores
    nw = nc * ns
    qpw = (B * N1) // nw
    wpb = nw // B
    nh = N2 // LANES
    mesh = plsc.VectorSubcoreMesh(core_axis_name="c", subcore_axis_name="s")

    @functools.partial(
        pl.kernel,
        mesh=mesh,
        out_type=jax.ShapeDtypeStruct((B * N1 // 2, 128), jnp.float32),
        compiler_params=pltpu.CompilerParams(needs_layout_passes=False),
        scratch_types=[
            pltpu.VMEM((qpw, nh), jnp.int32),
            pltpu.VMEM((qpw, 128), jnp.float32),
            pltpu.VMEM((nh + LANES,), jnp.int32),
            pltpu.VMEM((320,), jnp.int32),
            pltpu.VMEM((NSAMPLE,), jnp.int32),
            pltpu.VMEM((NSAMPLE,), jnp.int32),
            pltpu.VMEM((NSAMPLE, 128), jnp.float32),
            pltpu.VMEM((NSAMPLE, 128), jnp.float32),
            pltpu.VMEM((qpw // 2, 128), jnp.float32),
            pltpu.SemaphoreType.DMA,
            pltpu.SemaphoreType.DMA,
        ],
    )
    def kern(words_hbm, dq_hbm, t_hbm, out_hbm,
             words_v, dq_v, pos_v, list_v, idx0_v, idx1_v, rows0_v, rows1_v,
             out_v, sem0, sem1):
        wid = lax.axis_index("s") * nc + lax.axis_index("c")
        base = wid * qpw
        batch = wid // wpb
        pltpu.sync_copy(words_hbm.at[pl.ds(base, qpw)], words_v)
        pltpu.sync_copy(dq_hbm.at[pl.ds(base, qpw)], dq_v)

        lane_iota = lax.iota(jnp.int32, LANES)
        zero16 = jnp.zeros((LANES,), jnp.int32)
        tbase = batch * N2

        def popcnt(m):
            return plsc.all_reduce_population_count(m)[0]

        def scan_pad(i, idx_ref):
            list_v[pl.ds(0, LANES)] = zero16
            list_v[pl.ds(LANES, LANES)] = zero16

            def wgroup(g, wcnt):
                w = words_v[i, pl.ds(g * LANES, LANES)]
                m = w != 0
                plsc.store_compressed(
                    pos_v.at[pl.ds(wcnt, LANES)], lane_iota + g * LANES, mask=m)
                return wcnt + popcnt(m)

            wcnt = lax.fori_loop(0, nh // LANES, wgroup, 0, unroll=4)

            ivec = jnp.full((LANES,), i, jnp.int32)
            wcntv = jnp.full((LANES,), wcnt, jnp.int32)

            def block(state):
                j, cnt = state
                valid = lane_iota < (wcntv - jnp.full((LANES,), j, jnp.int32))
                posvec = jnp.where(valid, pos_v[pl.ds(j, LANES)], 0)
                vraw = plsc.load_gather(words_v, [ivec, posvec])
                c = jnp.where(valid, vraw >> 16, 0)
                w = vraw & 0xFFFF
                incl = plsc.cumsum(c)
                offs = incl - c + jnp.full((LANES,), cnt, jnp.int32)
                for l in range(LANES):
                    m = ((jnp.full((LANES,), w[l]) >> lane_iota) & 1) != 0
                    orig = jnp.full((LANES,), posvec[l] * LANES) + lane_iota
                    plsc.store_compressed(
                        list_v.at[pl.ds(offs[l], LANES)], orig, mask=m)
                return j + LANES, cnt + incl[LANES - 1]

            _, cnt = lax.while_loop(
                lambda s: (s[0] < wcnt) & (s[1] < NSAMPLE), block, (0, 0))

            first = list_v[pl.ds(0, LANES)][0]
            for h in range(2):
                cur = list_v[pl.ds(h * LANES, LANES)]
                lane = lane_iota + h * LANES
                idx_ref[pl.ds(h * LANES, LANES)] = (
                    jnp.where(lane < cnt, cur, first) + tbase)

        def compute(i, rows_v, half):
            dvec = [dq_v[i, pl.ds(h * LANES, LANES)] for h in range(4)]
            acc = [rows_v[0, pl.ds(h * LANES, LANES)]
                   - rows_v[0, pl.ds(64 + h * LANES, LANES)] * dvec[h]
                   for h in range(4)]
            for j in range(1, NSAMPLE):
                for h in range(4):
                    pj = rows_v[j, pl.ds(h * LANES, LANES)]
                    fj = rows_v[j, pl.ds(64 + h * LANES, LANES)]
                    acc[h] = jnp.maximum(acc[h], pj - fj * dvec[h])
            for h in range(4):
                out_v[i // 2, pl.ds(half + h * LANES, LANES)] = acc[h]

        def per_pair(k, carry):
            i0 = 2 * k
            i1 = 2 * k + 1
            scan_pad(i0, idx0_v)
            cp0 = pltpu.async_copy(t_hbm.at[idx0_v], rows0_v, sem0)
            scan_pad(i1, idx1_v)
            cp1 = pltpu.async_copy(t_hbm.at[idx1_v], rows1_v, sem1)
            cp0.wait()
            compute(i0, rows0_v, 0)
            cp1.wait()
            compute(i1, rows1_v, 64)
            return carry

        lax.fori_loop(0, qpw // 2, per_pair, 0)
        pltpu.sync_copy(out_v, out_hbm.at[pl.ds(wid * (qpw // 2), qpw // 2)])

    return kern(words, dq_flat, t_flat)


def kernel(query_xyz, support_xyz, query_mask, support_mask, support_features,
           W_conv, b_conv, bn_gamma, bn_beta):
    B, C, N2 = support_features.shape
    N1 = query_xyz.shape[1]

    params = jnp.zeros((8, 64), jnp.float32)
    params = params.at[0:3].set(W_conv.T / RADIUS).at[3].set(b_conv)
    params2 = jnp.zeros((8, 64), jnp.float32)
    params2 = params2.at[0].set(bn_gamma * (1.0 / math.sqrt(1.0 + 1e-5)))
    params2 = params2.at[1].set(bn_beta)

    feats_flat = support_features.reshape(B * C, N2)
    sxyz_flat = support_xyz.reshape(B * N2, 3)
    qxyz_flat = query_xyz.reshape(B * N1, 3)

    t_flat, dq_flat = _prep_call(feats_flat, sxyz_flat, qxyz_flat, params,
                                 B, N1, N2, C)

    sxyz_t = jnp.pad(jnp.swapaxes(support_xyz, 1, 2),
                     ((0, 0), (0, 5), (0, 0))).reshape(B * 8, N2)
    s_ar = jnp.arange(N2)
    grp = (s_ar[:, None] // LANES) == jnp.arange(N2 // LANES)[None, :]
    gmat = jnp.where(grp, (2.0 ** (s_ar % LANES))[:, None],
                     0.0).astype(jnp.bfloat16)
    gcmat = jnp.where(grp, 1.0, 0.0).astype(jnp.bfloat16)

    words = _mask_call(qxyz_flat, sxyz_t, gmat, gcmat, B, N1, N2)
    raw = _sc_call(words, dq_flat, t_flat, B, N1, N2)
    raw = raw.reshape(B * N1, 64)
    out = _post_call(raw, params2, B, N1, C)
    return out.reshape(B, C, N1)


# speedup vs baseline: 28.6498x; 1.0618x over previous
import functools
import math

import jax
import jax.numpy as jnp
from jax import lax
from jax.experimental import pallas as pl
from jax.experimental.pallas import tpu as pltpu
from jax.experimental.pallas import tpu_sc as plsc

RADIUS = 0.1
NSAMPLE = 32
LANES = 16
NB = 512


def _prep_body(feats_ref, sxyz_ref, qxyz_ref, p_ref, t_ref, d_ref):
    s = sxyz_ref[...]
    q = qxyz_ref[...]
    p = p_ref[...]
    a = (s[:, 0:1] * p[0:1, :] + s[:, 1:2] * p[1:2, :]
         + s[:, 2:3] * p[2:3, :] + p[3:4, :])
    d = (q[:, 0:1] * p[0:1, :] + q[:, 1:2] * p[1:2, :]
         + q[:, 2:3] * p[2:3, :])
    ft = jnp.transpose(feats_ref[...], (1, 0))
    t_ref[...] = jnp.concatenate([ft * a, ft], axis=1)
    d_ref[...] = d


def _prep_call(feats_flat, sxyz_flat, qxyz_flat, params, B, N1, N2, C):
    nblk = N2 // NB
    return pl.pallas_call(
        _prep_body,
        grid=(B, nblk),
        in_specs=[
            pl.BlockSpec((C, NB), lambda b, n: (b, n)),
            pl.BlockSpec((NB, 3), lambda b, n: (b * nblk + n, 0)),
            pl.BlockSpec((NB, 3), lambda b, n: (b * nblk + n, 0)),
            pl.BlockSpec((8, 64), lambda b, n: (0, 0)),
        ],
        out_specs=[
            pl.BlockSpec((NB, 128), lambda b, n: (b * nblk + n, 0)),
            pl.BlockSpec((NB, 64), lambda b, n: (b * nblk + n, 0)),
        ],
        out_shape=[
            jax.ShapeDtypeStruct((B * N2, 128), jnp.float32),
            jax.ShapeDtypeStruct((B * N1, 64), jnp.float32),
        ],
    )(feats_flat, sxyz_flat, qxyz_flat, params)


MB = 256


def _mask_body(qxyz_ref, sxyzt_ref, g_ref, gc_ref, w_ref):
    r2 = RADIUS * RADIUS
    q = qxyz_ref[...]
    dx = q[:, 0:1] - sxyzt_ref[0:1, :]
    dy = q[:, 1:2] - sxyzt_ref[1:2, :]
    dz = q[:, 2:3] - sxyzt_ref[2:3, :]
    d2 = (dx * dx + dy * dy) + dz * dz
    mb = jnp.where(d2 < r2, 1.0, 0.0).astype(jnp.bfloat16)
    dn = (((1,), (0,)), ((), ()))
    bits = jax.lax.dot_general(mb, g_ref[...], dn,
                               preferred_element_type=jnp.float32)
    cnts = jax.lax.dot_general(mb, gc_ref[...], dn,
                               preferred_element_type=jnp.float32)
    w_ref[...] = bits.astype(jnp.int32) + cnts.astype(jnp.int32) * 65536


def _mask_call(qxyz_flat, sxyz_t, gmat, gcmat, B, N1, N2):
    nblk = N1 // MB
    nh = N2 // LANES
    return pl.pallas_call(
        _mask_body,
        grid=(B, nblk),
        in_specs=[
            pl.BlockSpec((MB, 3), lambda b, n: (b * nblk + n, 0)),
            pl.BlockSpec((8, N2), lambda b, n: (b, 0)),
            pl.BlockSpec((N2, nh), lambda b, n: (0, 0)),
            pl.BlockSpec((N2, nh), lambda b, n: (0, 0)),
        ],
        out_specs=pl.BlockSpec((MB, nh), lambda b, n: (b * nblk + n, 0)),
        out_shape=jax.ShapeDtypeStruct((B * N1, nh), jnp.int32),
    )(qxyz_flat, sxyz_t, gmat, gcmat)


def _post_body(x_ref, p2_ref, o_ref):
    x = x_ref[...]
    y = jnp.maximum(x * p2_ref[0:1, :] + p2_ref[1:2, :], 0.0)
    o_ref[...] = jnp.transpose(y, (1, 0))


def _post_call(raw, params2, B, N1, C):
    nblk = N1 // NB
    return pl.pallas_call(
        _post_body,
        grid=(B, nblk),
        in_specs=[
            pl.BlockSpec((NB, C), lambda b, n: (b * nblk + n, 0)),
            pl.BlockSpec((8, 64), lambda b, n: (0, 0)),
        ],
        out_specs=pl.BlockSpec((C, NB), lambda b, n: (b, n)),
        out_shape=jax.ShapeDtypeStruct((B * C, N1), jnp.float32),
    )(raw, params2)


def _sc_call(words, dq_flat, t_flat, B, N1, N2):
    info = plsc.get_sparse_core_info()
    nc, ns = info.num_cores, info.num_subcores
    nw = nc * ns
    qpw = (B * N1) // nw
    wpb = nw // B
    nh = N2 // LANES
    mesh = plsc.VectorSubcoreMesh(core_axis_name="c", subcore_axis_name="s")

    @functools.partial(
        pl.kernel,
        mesh=mesh,
        out_type=jax.ShapeDtypeStruct((B * N1 // 2, 128), jnp.float32),
        compiler_params=pltpu.CompilerParams(needs_layout_passes=False),
        scratch_types=[
            pltpu.VMEM((qpw, nh), jnp.int32),
            pltpu.VMEM((qpw // 2, 128), jnp.float32),
            pltpu.VMEM((nh + LANES,), jnp.int32),
            pltpu.VMEM((320,), jnp.int32),
            pltpu.VMEM((NSAMPLE,), jnp.int32),
            pltpu.VMEM((NSAMPLE,), jnp.int32),
            pltpu.VMEM((NSAMPLE,), jnp.int32),
            pltpu.VMEM((NSAMPLE,), jnp.int32),
            pltpu.VMEM((NSAMPLE, 128), jnp.float32),
            pltpu.VMEM((NSAMPLE, 128), jnp.float32),
            pltpu.VMEM((NSAMPLE, 128), jnp.float32),
            pltpu.VMEM((NSAMPLE, 128), jnp.float32),
            pltpu.VMEM((qpw // 2, 128), jnp.float32),
            pltpu.SemaphoreType.DMA,
            pltpu.SemaphoreType.DMA,
            pltpu.SemaphoreType.DMA,
            pltpu.SemaphoreType.DMA,
        ],
    )
    def kern(words_hbm, dq_hbm, t_hbm, out_hbm,
             words_v, dq_v, pos_v, list_v, idx0_v, idx1_v, idx2_v, idx3_v,
             rows0_v, rows1_v, rows2_v, rows3_v,
             out_v, sem0, sem1, sem2, sem3):
        wid = lax.axis_index("s") * nc + lax.axis_index("c")
        base = wid * qpw
        batch = wid // wpb
        pltpu.sync_copy(words_hbm.at[pl.ds(base, qpw)], words_v)
        pltpu.sync_copy(
            dq_hbm.at[pl.ds(pl.multiple_of(base // 2, 8), qpw // 2)], dq_v)

        lane_iota = lax.iota(jnp.int32, LANES)
        zero16 = jnp.zeros((LANES,), jnp.int32)
        tbase = batch * N2

        def popcnt(m):
            return plsc.all_reduce_population_count(m)[0]

        def scan_pad(i, idx_ref):
            list_v[pl.ds(0, LANES)] = zero16
            list_v[pl.ds(LANES, LANES)] = zero16

            def wgroup(g, wcnt):
                w = words_v[i, pl.ds(g * LANES, LANES)]
                m = w != 0
                plsc.store_compressed(
                    pos_v.at[pl.ds(wcnt, LANES)], lane_iota + g * LANES, mask=m)
                return wcnt + popcnt(m)

            wcnt = lax.fori_loop(0, nh // LANES, wgroup, 0, unroll=4)

            ivec = jnp.full((LANES,), i, jnp.int32)
            wcntv = jnp.full((LANES,), wcnt, jnp.int32)

            def block(state):
                j, cnt = state
                valid = lane_iota < (wcntv - jnp.full((LANES,), j, jnp.int32))
                posvec = jnp.where(valid, pos_v[pl.ds(j, LANES)], 0)
                vraw = plsc.load_gather(words_v, [ivec, posvec])
                c = jnp.where(valid, vraw >> 16, 0)
                w = vraw & 0xFFFF
                incl = plsc.cumsum(c)
                offs = incl - c + jnp.full((LANES,), cnt, jnp.int32)
                for l in range(LANES):
                    m = ((jnp.full((LANES,), w[l]) >> lane_iota) & 1) != 0
                    orig = jnp.full((LANES,), posvec[l] * LANES) + lane_iota
                    plsc.store_compressed(
                        list_v.at[pl.ds(offs[l], LANES)], orig, mask=m)
                return j + LANES, cnt + incl[LANES - 1]

            _, cnt = lax.while_loop(
                lambda s: (s[0] < wcnt) & (s[1] < NSAMPLE), block, (0, 0))

            first = list_v[pl.ds(0, LANES)][0]
            for h in range(2):
                cur = list_v[pl.ds(h * LANES, LANES)]
                lane = lane_iota + h * LANES
                idx_ref[pl.ds(h * LANES, LANES)] = (
                    jnp.where(lane < cnt, cur, first) + tbase)
            return cnt

        def compute(row, half, rows_v):
            dvec = [dq_v[row, pl.ds(half + h * LANES, LANES)] for h in range(4)]
            acc = [rows_v[0, pl.ds(h * LANES, LANES)]
                   - rows_v[0, pl.ds(64 + h * LANES, LANES)] * dvec[h]
                   for h in range(4)]
            for j in range(1, NSAMPLE):
                for h in range(4):
                    pj = rows_v[j, pl.ds(h * LANES, LANES)]
                    fj = rows_v[j, pl.ds(64 + h * LANES, LANES)]
                    acc[h] = jnp.maximum(acc[h], pj - fj * dvec[h])
            for h in range(4):
                out_v[row, pl.ds(half + h * LANES, LANES)] = acc[h]

        idx_bufs = (idx0_v, idx1_v, idx2_v, idx3_v)
        row_bufs = (rows0_v, rows1_v, rows2_v, rows3_v)
        sems = (sem0, sem1, sem2, sem3)

        def per_quad(k, carry):
            cps = []
            for j in range(4):
                scan_pad(4 * k + j, idx_bufs[j])
                cps.append(pltpu.async_copy(
                    t_hbm.at[idx_bufs[j]], row_bufs[j], sems[j]))
            for j in range(4):
                cps[j].wait()
                compute(2 * k + j // 2, (j % 2) * 64, row_bufs[j])
            return carry

        lax.fori_loop(0, qpw // 4, per_quad, 0)
        pltpu.sync_copy(out_v, out_hbm.at[pl.ds(wid * (qpw // 2), qpw // 2)])

    return kern(words, dq_flat, t_flat)


def kernel(query_xyz, support_xyz, query_mask, support_mask, support_features,
           W_conv, b_conv, bn_gamma, bn_beta):
    B, C, N2 = support_features.shape
    N1 = query_xyz.shape[1]

    params = jnp.zeros((8, 64), jnp.float32)
    params = params.at[0:3].set(W_conv.T / RADIUS).at[3].set(b_conv)
    params2 = jnp.zeros((8, 64), jnp.float32)
    params2 = params2.at[0].set(bn_gamma * (1.0 / math.sqrt(1.0 + 1e-5)))
    params2 = params2.at[1].set(bn_beta)

    feats_flat = support_features.reshape(B * C, N2)
    sxyz_flat = support_xyz.reshape(B * N2, 3)
    qxyz_flat = query_xyz.reshape(B * N1, 3)

    t_flat, dq_flat = _prep_call(feats_flat, sxyz_flat, qxyz_flat, params,
                                 B, N1, N2, C)

    sxyz_t = jnp.pad(jnp.swapaxes(support_xyz, 1, 2),
                     ((0, 0), (0, 5), (0, 0))).reshape(B * 8, N2)
    s_ar = jnp.arange(N2)
    grp = (s_ar[:, None] // LANES) == jnp.arange(N2 // LANES)[None, :]
    gmat = jnp.where(grp, (2.0 ** (s_ar % LANES))[:, None],
                     0.0).astype(jnp.bfloat16)
    gcmat = jnp.where(grp, 1.0, 0.0).astype(jnp.bfloat16)

    words = _mask_call(qxyz_flat, sxyz_t, gmat, gcmat, B, N1, N2)
    raw = _sc_call(words, dq_flat.reshape(B * N1 // 2, 128), t_flat, B, N1, N2)
    raw = raw.reshape(B * N1, 64)
    out = _post_call(raw, params2, B, N1, C)
    return out.reshape(B, C, N1)


# speedup vs baseline: 34.2280x; 1.1947x over previous
import functools
import math

import jax
import jax.numpy as jnp
from jax import lax
from jax.experimental import pallas as pl
from jax.experimental.pallas import tpu as pltpu
from jax.experimental.pallas import tpu_sc as plsc

RADIUS = 0.1
NSAMPLE = 32
LANES = 16
NB = 512


def _prep_body(feats_ref, sxyz_ref, qxyz_ref, p_ref, t_ref, d_ref):
    s = sxyz_ref[...]
    q = qxyz_ref[...]
    p = p_ref[...]
    a = (s[:, 0:1] * p[0:1, :] + s[:, 1:2] * p[1:2, :]
         + s[:, 2:3] * p[2:3, :] + p[3:4, :])
    d = (q[:, 0:1] * p[0:1, :] + q[:, 1:2] * p[1:2, :]
         + q[:, 2:3] * p[2:3, :])
    ft = jnp.transpose(feats_ref[...], (1, 0))
    t_ref[...] = jnp.concatenate([ft * a, ft], axis=1)
    d_ref[...] = d


def _prep_call(feats_flat, sxyz_flat, qxyz_flat, params, B, N1, N2, C):
    nblk = N2 // NB
    return pl.pallas_call(
        _prep_body,
        grid=(B, nblk),
        in_specs=[
            pl.BlockSpec((C, NB), lambda b, n: (b, n)),
            pl.BlockSpec((NB, 3), lambda b, n: (b * nblk + n, 0)),
            pl.BlockSpec((NB, 3), lambda b, n: (b * nblk + n, 0)),
            pl.BlockSpec((8, 64), lambda b, n: (0, 0)),
        ],
        out_specs=[
            pl.BlockSpec((NB, 128), lambda b, n: (b * nblk + n, 0)),
            pl.BlockSpec((NB, 64), lambda b, n: (b * nblk + n, 0)),
        ],
        out_shape=[
            jax.ShapeDtypeStruct((B * N2, 128), jnp.float32),
            jax.ShapeDtypeStruct((B * N1, 64), jnp.float32),
        ],
    )(feats_flat, sxyz_flat, qxyz_flat, params)


MB = 256


def _mask_body(qxyz_ref, sxyzt_ref, g_ref, gc_ref, w_ref):
    r2 = RADIUS * RADIUS
    q = qxyz_ref[...]
    dx = q[:, 0:1] - sxyzt_ref[0:1, :]
    dy = q[:, 1:2] - sxyzt_ref[1:2, :]
    dz = q[:, 2:3] - sxyzt_ref[2:3, :]
    d2 = (dx * dx + dy * dy) + dz * dz
    mb = jnp.where(d2 < r2, 1.0, 0.0).astype(jnp.bfloat16)
    dn = (((1,), (0,)), ((), ()))
    bits = jax.lax.dot_general(mb, g_ref[...], dn,
                               preferred_element_type=jnp.float32)
    cnts = jax.lax.dot_general(mb, gc_ref[...], dn,
                               preferred_element_type=jnp.float32)
    w_ref[...] = bits.astype(jnp.int32) + cnts.astype(jnp.int32) * 65536


def _mask_call(qxyz_flat, sxyz_t, gmat, gcmat, B, N1, N2):
    nblk = N1 // MB
    nh = N2 // LANES
    return pl.pallas_call(
        _mask_body,
        grid=(B, nblk),
        in_specs=[
            pl.BlockSpec((MB, 3), lambda b, n: (b * nblk + n, 0)),
            pl.BlockSpec((8, N2), lambda b, n: (b, 0)),
            pl.BlockSpec((N2, nh), lambda b, n: (0, 0)),
            pl.BlockSpec((N2, nh), lambda b, n: (0, 0)),
        ],
        out_specs=pl.BlockSpec((MB, nh), lambda b, n: (b * nblk + n, 0)),
        out_shape=jax.ShapeDtypeStruct((B * N1, nh), jnp.int32),
    )(qxyz_flat, sxyz_t, gmat, gcmat)


def _post_body(x_ref, p2_ref, o_ref):
    x = x_ref[...]
    y = jnp.maximum(x * p2_ref[0:1, :] + p2_ref[1:2, :], 0.0)
    o_ref[...] = jnp.transpose(y, (1, 0))


def _post_call(raw, params2, B, N1, C):
    nblk = N1 // NB
    return pl.pallas_call(
        _post_body,
        grid=(B, nblk),
        in_specs=[
            pl.BlockSpec((NB, C), lambda b, n: (b * nblk + n, 0)),
            pl.BlockSpec((8, 64), lambda b, n: (0, 0)),
        ],
        out_specs=pl.BlockSpec((C, NB), lambda b, n: (b, n)),
        out_shape=jax.ShapeDtypeStruct((B * C, N1), jnp.float32),
    )(raw, params2)


def _sc_call(words, dq_flat, t_flat, B, N1, N2):
    info = plsc.get_sparse_core_info()
    nc, ns = info.num_cores, info.num_subcores
    nw = nc * ns
    qpw = (B * N1) // nw
    wpb = nw // B
    nh = N2 // LANES
    mesh = plsc.VectorSubcoreMesh(core_axis_name="c", subcore_axis_name="s")

    @functools.partial(
        pl.kernel,
        mesh=mesh,
        out_type=jax.ShapeDtypeStruct((B * N1 // 2, 128), jnp.float32),
        compiler_params=pltpu.CompilerParams(needs_layout_passes=False),
        scratch_types=[
            pltpu.VMEM((qpw, nh), jnp.int32),
            pltpu.VMEM((qpw // 2, 128), jnp.float32),
            pltpu.VMEM((nh + LANES,), jnp.int32),
            pltpu.VMEM((320,), jnp.int32),
            pltpu.VMEM((LANES,), jnp.int32),
            pltpu.VMEM((LANES,), jnp.int32),
            pltpu.VMEM((LANES,), jnp.int32),
            pltpu.VMEM((LANES,), jnp.int32),
            pltpu.VMEM((LANES,), jnp.int32),
            pltpu.VMEM((LANES,), jnp.int32),
            pltpu.VMEM((LANES,), jnp.int32),
            pltpu.VMEM((LANES,), jnp.int32),
            pltpu.VMEM((LANES, 128), jnp.float32),
            pltpu.VMEM((LANES, 128), jnp.float32),
            pltpu.VMEM((LANES, 128), jnp.float32),
            pltpu.VMEM((LANES, 128), jnp.float32),
            pltpu.VMEM((LANES, 128), jnp.float32),
            pltpu.VMEM((LANES, 128), jnp.float32),
            pltpu.VMEM((LANES, 128), jnp.float32),
            pltpu.VMEM((LANES, 128), jnp.float32),
            pltpu.VMEM((qpw // 2, 128), jnp.float32),
            pltpu.SemaphoreType.DMA,
            pltpu.SemaphoreType.DMA,
            pltpu.SemaphoreType.DMA,
            pltpu.SemaphoreType.DMA,
            pltpu.SemaphoreType.DMA,
            pltpu.SemaphoreType.DMA,
            pltpu.SemaphoreType.DMA,
            pltpu.SemaphoreType.DMA,
        ],
    )
    def kern(words_hbm, dq_hbm, t_hbm, out_hbm,
             words_v, dq_v, pos_v, list_v,
             ilo0, ihi0, ilo1, ihi1, ilo2, ihi2, ilo3, ihi3,
             rlo0, rhi0, rlo1, rhi1, rlo2, rhi2, rlo3, rhi3,
             out_v, slo0, shi0, slo1, shi1, slo2, shi2, slo3, shi3):
        wid = lax.axis_index("s") * nc + lax.axis_index("c")
        base = wid * qpw
        batch = wid // wpb
        pltpu.sync_copy(words_hbm.at[pl.ds(base, qpw)], words_v)
        pltpu.sync_copy(
            dq_hbm.at[pl.ds(pl.multiple_of(base // 2, 8), qpw // 2)], dq_v)

        lane_iota = lax.iota(jnp.int32, LANES)
        zero16 = jnp.zeros((LANES,), jnp.int32)
        tbase = batch * N2

        def popcnt(m):
            return plsc.all_reduce_population_count(m)[0]

        def scan_pad(i, idxlo_ref, idxhi_ref):
            list_v[pl.ds(0, LANES)] = zero16
            list_v[pl.ds(LANES, LANES)] = zero16

            def wgroup(g, wcnt):
                w = words_v[i, pl.ds(g * LANES, LANES)]
                m = w != 0
                plsc.store_compressed(
                    pos_v.at[pl.ds(wcnt, LANES)], lane_iota + g * LANES, mask=m)
                return wcnt + popcnt(m)

            wcnt = lax.fori_loop(0, nh // LANES, wgroup, 0, unroll=4)

            ivec = jnp.full((LANES,), i, jnp.int32)
            wcntv = jnp.full((LANES,), wcnt, jnp.int32)

            def block(state):
                j, cnt = state
                valid = lane_iota < (wcntv - jnp.full((LANES,), j, jnp.int32))
                posvec = jnp.where(valid, pos_v[pl.ds(j, LANES)], 0)
                vraw = plsc.load_gather(words_v, [ivec, posvec])
                c = jnp.where(valid, vraw >> 16, 0)
                w = vraw & 0xFFFF
                incl = plsc.cumsum(c)
                offs = incl - c + jnp.full((LANES,), cnt, jnp.int32)
                for l in range(LANES):
                    m = ((jnp.full((LANES,), w[l]) >> lane_iota) & 1) != 0
                    orig = jnp.full((LANES,), posvec[l] * LANES) + lane_iota
                    plsc.store_compressed(
                        list_v.at[pl.ds(offs[l], LANES)], orig, mask=m)
                return j + LANES, cnt + incl[LANES - 1]

            _, cnt = lax.while_loop(
                lambda s: (s[0] < wcnt) & (s[1] < NSAMPLE), block, (0, 0))

            first = list_v[pl.ds(0, LANES)][0]
            for h, idx_ref in ((0, idxlo_ref), (1, idxhi_ref)):
                cur = list_v[pl.ds(h * LANES, LANES)]
                lane = lane_iota + h * LANES
                idx_ref[...] = jnp.where(lane < cnt, cur, first) + tbase
            return cnt

        def max_rows(rows_v, dvec, acc):
            for j in range(LANES):
                for h in range(4):
                    pj = rows_v[j, pl.ds(h * LANES, LANES)]
                    fj = rows_v[j, pl.ds(64 + h * LANES, LANES)]
                    a = pj - fj * dvec[h]
                    acc[h] = a if acc[h] is None else jnp.maximum(acc[h], a)
            return acc

        idx_bufs = ((ilo0, ihi0), (ilo1, ihi1), (ilo2, ihi2), (ilo3, ihi3))
        row_bufs = ((rlo0, rhi0), (rlo1, rhi1), (rlo2, rhi2), (rlo3, rhi3))
        sems = ((slo0, shi0), (slo1, shi1), (slo2, shi2), (slo3, shi3))

        def per_quad(k, carry):
            cps = []
            cnts = []
            for j in range(4):
                cnt = scan_pad(4 * k + j, idx_bufs[j][0], idx_bufs[j][1])
                cps.append(pltpu.async_copy(
                    t_hbm.at[idx_bufs[j][0]], row_bufs[j][0], sems[j][0]))

                @pl.when(cnt > LANES)
                def _(j=j):
                    pltpu.async_copy(
                        t_hbm.at[idx_bufs[j][1]], row_bufs[j][1], sems[j][1])

                cnts.append(cnt)
            for j in range(4):
                row = 2 * k + j // 2
                half = (j % 2) * 64
                dvec = [dq_v[row, pl.ds(half + h * LANES, LANES)]
                        for h in range(4)]
                cps[j].wait()
                acc = max_rows(row_bufs[j][0], dvec, [None] * 4)

                def with_hi(j=j, dvec=dvec, acc=acc):
                    pltpu.make_async_copy(
                        t_hbm.at[idx_bufs[j][1]], row_bufs[j][1],
                        sems[j][1]).wait()
                    return tuple(max_rows(row_bufs[j][1], dvec, list(acc)))

                acc = lax.cond(cnts[j] > LANES, with_hi,
                               lambda acc=acc: tuple(acc))
                for h in range(4):
                    out_v[row, pl.ds(half + h * LANES, LANES)] = acc[h]
            return carry

        lax.fori_loop(0, qpw // 4, per_quad, 0)
        pltpu.sync_copy(out_v, out_hbm.at[pl.ds(wid * (qpw // 2), qpw // 2)])

    return kern(words, dq_flat, t_flat)


def kernel(query_xyz, support_xyz, query_mask, support_mask, support_features,
           W_conv, b_conv, bn_gamma, bn_beta):
    B, C, N2 = support_features.shape
    N1 = query_xyz.shape[1]

    params = jnp.zeros((8, 64), jnp.float32)
    params = params.at[0:3].set(W_conv.T / RADIUS).at[3].set(b_conv)
    params2 = jnp.zeros((8, 64), jnp.float32)
    params2 = params2.at[0].set(bn_gamma * (1.0 / math.sqrt(1.0 + 1e-5)))
    params2 = params2.at[1].set(bn_beta)

    feats_flat = support_features.reshape(B * C, N2)
    sxyz_flat = support_xyz.reshape(B * N2, 3)
    qxyz_flat = query_xyz.reshape(B * N1, 3)

    t_flat, dq_flat = _prep_call(feats_flat, sxyz_flat, qxyz_flat, params,
                                 B, N1, N2, C)

    sxyz_t = jnp.pad(jnp.swapaxes(support_xyz, 1, 2),
                     ((0, 0), (0, 5), (0, 0))).reshape(B * 8, N2)
    s_ar = jnp.arange(N2)
    grp = (s_ar[:, None] // LANES) == jnp.arange(N2 // LANES)[None, :]
    gmat = jnp.where(grp, (2.0 ** (s_ar % LANES))[:, None],
                     0.0).astype(jnp.bfloat16)
    gcmat = jnp.where(grp, 1.0, 0.0).astype(jnp.bfloat16)

    words = _mask_call(qxyz_flat, sxyz_t, gmat, gcmat, B, N1, N2)
    raw = _sc_call(words, dq_flat.reshape(B * N1 // 2, 128), t_flat, B, N1, N2)
    raw = raw.reshape(B * N1, 64)
    out = _post_call(raw, params2, B, N1, C)
    return out.reshape(B, C, N1)


# speedup vs baseline: 38.5894x; 1.1274x over previous
import functools
import math

import jax
import jax.numpy as jnp
from jax import lax
from jax.experimental import pallas as pl
from jax.experimental.pallas import tpu as pltpu
from jax.experimental.pallas import tpu_sc as plsc

RADIUS = 0.1
NSAMPLE = 32
LANES = 16
NB = 512


def _prep_body(feats_ref, sxyz_ref, qxyz_ref, p_ref, t_ref, d_ref):
    s = sxyz_ref[...]
    q = qxyz_ref[...]
    p = p_ref[...]
    a = (s[:, 0:1] * p[0:1, :] + s[:, 1:2] * p[1:2, :]
         + s[:, 2:3] * p[2:3, :] + p[3:4, :])
    d = (q[:, 0:1] * p[0:1, :] + q[:, 1:2] * p[1:2, :]
         + q[:, 2:3] * p[2:3, :])
    ft = jnp.transpose(feats_ref[...], (1, 0))
    t_ref[...] = jnp.concatenate([ft * a, ft], axis=1)
    d_ref[...] = d


def _prep_call(feats_flat, sxyz_flat, qxyz_flat, params, B, N1, N2, C):
    nblk = N2 // NB
    return pl.pallas_call(
        _prep_body,
        grid=(B, nblk),
        in_specs=[
            pl.BlockSpec((C, NB), lambda b, n: (b, n)),
            pl.BlockSpec((NB, 3), lambda b, n: (b * nblk + n, 0)),
            pl.BlockSpec((NB, 3), lambda b, n: (b * nblk + n, 0)),
            pl.BlockSpec((8, 64), lambda b, n: (0, 0)),
        ],
        out_specs=[
            pl.BlockSpec((NB, 128), lambda b, n: (b * nblk + n, 0)),
            pl.BlockSpec((NB, 64), lambda b, n: (b * nblk + n, 0)),
        ],
        out_shape=[
            jax.ShapeDtypeStruct((B * N2, 128), jnp.float32),
            jax.ShapeDtypeStruct((B * N1, 64), jnp.float32),
        ],
    )(feats_flat, sxyz_flat, qxyz_flat, params)


MB = 256


def _mask_body(qxyz_ref, sxyzt_ref, g_ref, gc_ref, w_ref):
    r2 = RADIUS * RADIUS
    q = qxyz_ref[...]
    dx = q[:, 0:1] - sxyzt_ref[0:1, :]
    dy = q[:, 1:2] - sxyzt_ref[1:2, :]
    dz = q[:, 2:3] - sxyzt_ref[2:3, :]
    d2 = (dx * dx + dy * dy) + dz * dz
    mb = jnp.where(d2 < r2, 1.0, 0.0).astype(jnp.bfloat16)
    dn = (((1,), (0,)), ((), ()))
    bits = jax.lax.dot_general(mb, g_ref[...], dn,
                               preferred_element_type=jnp.float32)
    cnts = jax.lax.dot_general(mb, gc_ref[...], dn,
                               preferred_element_type=jnp.float32)
    w_ref[...] = bits.astype(jnp.int32) + cnts.astype(jnp.int32) * 65536


def _mask_call(qxyz_flat, sxyz_t, gmat, gcmat, B, N1, N2):
    nblk = N1 // MB
    nh = N2 // LANES
    return pl.pallas_call(
        _mask_body,
        grid=(B, nblk),
        in_specs=[
            pl.BlockSpec((MB, 3), lambda b, n: (b * nblk + n, 0)),
            pl.BlockSpec((8, N2), lambda b, n: (b, 0)),
            pl.BlockSpec((N2, nh), lambda b, n: (0, 0)),
            pl.BlockSpec((N2, nh), lambda b, n: (0, 0)),
        ],
        out_specs=pl.BlockSpec((MB, nh), lambda b, n: (b * nblk + n, 0)),
        out_shape=jax.ShapeDtypeStruct((B * N1, nh), jnp.int32),
    )(qxyz_flat, sxyz_t, gmat, gcmat)


def _post_body(x_ref, p2_ref, o_ref):
    x = x_ref[...]
    y = jnp.maximum(x * p2_ref[0:1, :] + p2_ref[1:2, :], 0.0)
    o_ref[...] = jnp.transpose(y, (1, 0))


def _post_call(raw, params2, B, N1, C):
    nblk = N1 // NB
    return pl.pallas_call(
        _post_body,
        grid=(B, nblk),
        in_specs=[
            pl.BlockSpec((NB, C), lambda b, n: (b * nblk + n, 0)),
            pl.BlockSpec((8, 64), lambda b, n: (0, 0)),
        ],
        out_specs=pl.BlockSpec((C, NB), lambda b, n: (b, n)),
        out_shape=jax.ShapeDtypeStruct((B * C, N1), jnp.float32),
    )(raw, params2)


def _sc_call(words, dq_flat, t_flat, B, N1, N2):
    info = plsc.get_sparse_core_info()
    nc, ns = info.num_cores, info.num_subcores
    nw = nc * ns
    qpw = (B * N1) // nw
    wpb = nw // B
    nh = N2 // LANES
    mesh = plsc.VectorSubcoreMesh(core_axis_name="c", subcore_axis_name="s")

    @functools.partial(
        pl.kernel,
        mesh=mesh,
        out_type=jax.ShapeDtypeStruct((B * N1 // 2, 128), jnp.float32),
        compiler_params=pltpu.CompilerParams(needs_layout_passes=False),
        scratch_types=[
            pltpu.VMEM((qpw, nh), jnp.int32),
            pltpu.VMEM((qpw // 2, 128), jnp.float32),
            pltpu.VMEM((nh + LANES,), jnp.int32),
            pltpu.VMEM((320,), jnp.int32),
            pltpu.VMEM((NSAMPLE,), jnp.int32),
            pltpu.VMEM((NSAMPLE,), jnp.int32),
            pltpu.VMEM((NSAMPLE,), jnp.int32),
            pltpu.VMEM((NSAMPLE,), jnp.int32),
            pltpu.VMEM((NSAMPLE, 128), jnp.float32),
            pltpu.VMEM((NSAMPLE, 128), jnp.float32),
            pltpu.VMEM((NSAMPLE, 128), jnp.float32),
            pltpu.VMEM((NSAMPLE, 128), jnp.float32),
            pltpu.VMEM((qpw // 2, 128), jnp.float32),
            pltpu.SemaphoreType.DMA,
            pltpu.SemaphoreType.DMA,
            pltpu.SemaphoreType.DMA,
            pltpu.SemaphoreType.DMA,
            pltpu.SemaphoreType.DMA,
            pltpu.SemaphoreType.DMA,
            pltpu.SemaphoreType.DMA,
            pltpu.SemaphoreType.DMA,
            pltpu.SemaphoreType.DMA,
            pltpu.SemaphoreType.DMA,
            pltpu.SemaphoreType.DMA,
            pltpu.SemaphoreType.DMA,
            pltpu.SemaphoreType.DMA,
            pltpu.SemaphoreType.DMA,
            pltpu.SemaphoreType.DMA,
            pltpu.SemaphoreType.DMA,
        ],
    )
    def kern(words_hbm, dq_hbm, t_hbm, out_hbm,
             words_v, dq_v, pos_v, list_v,
             idx0_v, idx1_v, idx2_v, idx3_v,
             rows0_v, rows1_v, rows2_v, rows3_v, out_v,
             s00, s01, s02, s03, s10, s11, s12, s13,
             s20, s21, s22, s23, s30, s31, s32, s33):
        wid = lax.axis_index("s") * nc + lax.axis_index("c")
        base = wid * qpw
        batch = wid // wpb
        pltpu.sync_copy(words_hbm.at[pl.ds(base, qpw)], words_v)
        pltpu.sync_copy(
            dq_hbm.at[pl.ds(pl.multiple_of(base // 2, 8), qpw // 2)], dq_v)

        lane_iota = lax.iota(jnp.int32, LANES)
        zero16 = jnp.zeros((LANES,), jnp.int32)
        tbase = batch * N2

        def popcnt(m):
            return plsc.all_reduce_population_count(m)[0]

        def scan_pad(i, idx_ref):
            list_v[pl.ds(0, LANES)] = zero16
            list_v[pl.ds(LANES, LANES)] = zero16

            def wgroup(g, wcnt):
                w = words_v[i, pl.ds(g * LANES, LANES)]
                m = w != 0
                plsc.store_compressed(
                    pos_v.at[pl.ds(wcnt, LANES)], lane_iota + g * LANES, mask=m)
                return wcnt + popcnt(m)

            wcnt = lax.fori_loop(0, nh // LANES, wgroup, 0, unroll=4)

            ivec = jnp.full((LANES,), i, jnp.int32)
            wcntv = jnp.full((LANES,), wcnt, jnp.int32)

            def block(state):
                j, cnt = state
                valid = lane_iota < (wcntv - jnp.full((LANES,), j, jnp.int32))
                posvec = jnp.where(valid, pos_v[pl.ds(j, LANES)], 0)
                vraw = plsc.load_gather(words_v, [ivec, posvec])
                c = jnp.where(valid, vraw >> 16, 0)
                w = vraw & 0xFFFF
                incl = plsc.cumsum(c)
                offs = incl - c + jnp.full((LANES,), cnt, jnp.int32)
                for l in range(LANES):
                    m = ((jnp.full((LANES,), w[l]) >> lane_iota) & 1) != 0
                    orig = jnp.full((LANES,), posvec[l] * LANES) + lane_iota
                    plsc.store_compressed(
                        list_v.at[pl.ds(offs[l], LANES)], orig, mask=m)
                return j + LANES, cnt + incl[LANES - 1]

            _, cnt = lax.while_loop(
                lambda s: (s[0] < wcnt) & (s[1] < NSAMPLE), block, (0, 0))

            first = list_v[pl.ds(0, LANES)][0]
            for h in range(2):
                cur = list_v[pl.ds(h * LANES, LANES)]
                lane = lane_iota + h * LANES
                idx_ref[pl.ds(h * LANES, LANES)] = (
                    jnp.where(lane < cnt, cur, first) + tbase)
            return cnt

        Q8 = 8

        def max_rows(rows_v, q, dvec, acc):
            for j in range(q * Q8, q * Q8 + Q8):
                for h in range(4):
                    pj = rows_v[j, pl.ds(h * LANES, LANES)]
                    fj = rows_v[j, pl.ds(64 + h * LANES, LANES)]
                    a = pj - fj * dvec[h]
                    acc[h] = a if acc[h] is None else jnp.maximum(acc[h], a)
            return acc

        idx_bufs = (idx0_v, idx1_v, idx2_v, idx3_v)
        row_bufs = (rows0_v, rows1_v, rows2_v, rows3_v)
        sems = ((s00, s01, s02, s03), (s10, s11, s12, s13),
                (s20, s21, s22, s23), (s30, s31, s32, s33))

        def per_quad(k, carry):
            cnts = []
            for j in range(4):
                cnt = scan_pad(4 * k + j, idx_bufs[j])
                pltpu.async_copy(
                    t_hbm.at[idx_bufs[j].at[pl.ds(0, Q8)]],
                    row_bufs[j].at[pl.ds(0, Q8)], sems[j][0])
                for q in range(1, 4):
                    @pl.when(cnt > q * Q8)
                    def _(j=j, q=q):
                        pltpu.async_copy(
                            t_hbm.at[idx_bufs[j].at[pl.ds(q * Q8, Q8)]],
                            row_bufs[j].at[pl.ds(q * Q8, Q8)], sems[j][q])

                cnts.append(cnt)
            for j in range(4):
                row = 2 * k + j // 2
                half = (j % 2) * 64
                dvec = [dq_v[row, pl.ds(half + h * LANES, LANES)]
                        for h in range(4)]
                pltpu.make_async_copy(
                    t_hbm.at[idx_bufs[j].at[pl.ds(0, Q8)]],
                    row_bufs[j].at[pl.ds(0, Q8)], sems[j][0]).wait()
                acc = max_rows(row_bufs[j], 0, dvec, [None] * 4)
                for q in range(1, 4):
                    def with_q(j=j, q=q, dvec=dvec, acc=acc):
                        pltpu.make_async_copy(
                            t_hbm.at[idx_bufs[j].at[pl.ds(q * Q8, Q8)]],
                            row_bufs[j].at[pl.ds(q * Q8, Q8)],
                            sems[j][q]).wait()
                        return tuple(max_rows(row_bufs[j], q, dvec, list(acc)))

                    acc = list(lax.cond(cnts[j] > q * Q8, with_q,
                                        lambda acc=acc: tuple(acc)))
                for h in range(4):
                    out_v[row, pl.ds(half + h * LANES, LANES)] = acc[h]
            return carry

        lax.fori_loop(0, qpw // 4, per_quad, 0)
        pltpu.sync_copy(out_v, out_hbm.at[pl.ds(wid * (qpw // 2), qpw // 2)])

    return kern(words, dq_flat, t_flat)


def kernel(query_xyz, support_xyz, query_mask, support_mask, support_features,
           W_conv, b_conv, bn_gamma, bn_beta):
    B, C, N2 = support_features.shape
    N1 = query_xyz.shape[1]

    params = jnp.zeros((8, 64), jnp.float32)
    params = params.at[0:3].set(W_conv.T / RADIUS).at[3].set(b_conv)
    params2 = jnp.zeros((8, 64), jnp.float32)
    params2 = params2.at[0].set(bn_gamma * (1.0 / math.sqrt(1.0 + 1e-5)))
    params2 = params2.at[1].set(bn_beta)

    feats_flat = support_features.reshape(B * C, N2)
    sxyz_flat = support_xyz.reshape(B * N2, 3)
    qxyz_flat = query_xyz.reshape(B * N1, 3)

    t_flat, dq_flat = _prep_call(feats_flat, sxyz_flat, qxyz_flat, params,
                                 B, N1, N2, C)

    sxyz_t = jnp.pad(jnp.swapaxes(support_xyz, 1, 2),
                     ((0, 0), (0, 5), (0, 0))).reshape(B * 8, N2)
    s_ar = jnp.arange(N2)
    grp = (s_ar[:, None] // LANES) == jnp.arange(N2 // LANES)[None, :]
    gmat = jnp.where(grp, (2.0 ** (s_ar % LANES))[:, None],
                     0.0).astype(jnp.bfloat16)
    gcmat = jnp.where(grp, 1.0, 0.0).astype(jnp.bfloat16)

    words = _mask_call(qxyz_flat, sxyz_t, gmat, gcmat, B, N1, N2)
    raw = _sc_call(words, dq_flat.reshape(B * N1 // 2, 128), t_flat, B, N1, N2)
    raw = raw.reshape(B * N1, 64)
    out = _post_call(raw, params2, B, N1, C)
    return out.reshape(B, C, N1)
